# baseline (speedup 1.0000x reference)
import functools

import jax
import jax.numpy as jnp
from jax.experimental import pallas as pl
from jax.experimental.pallas import tpu as pltpu


def _pick_block(n, target=512):
    for bm in (512, 400, 256, 200, 128, 80, 64, 40, 32, 16, 8):
        if bm <= target and n % bm == 0:
            return bm
    return n


def _body(x_ref, adj_ref, w1_hbm, b1_hbm, w11_hbm, b11_hbm,
          w2_hbm, b2_hbm, w21_hbm, b21_hbm, o_ref,
          s_ref, t_ref, h_ref,
          w1_v, b1_v, w11_v, b11_v, w2_v, b2_v, w21_v, b21_v,
          sem1, sem2, *, bm, nj, hdim, cdim):
    t = pl.program_id(0)

    layer1 = ((w1_hbm, w1_v, 0), (b1_hbm, b1_v, 1),
              (w11_hbm, w11_v, 2), (b11_hbm, b11_v, 3))
    layer2 = ((w2_hbm, w2_v, 0), (b2_hbm, b2_v, 1),
              (w21_hbm, w21_v, 2), (b21_hbm, b21_v, 3))

    @pl.when(t == 0)
    def _():
        for src, dst, i in layer2:
            pltpu.make_async_copy(src, dst, sem2.at[i]).start()
        for src, dst, i in layer1:
            pltpu.make_async_copy(src, dst, sem1.at[i]).start()
        for src, dst, i in layer1:
            pltpu.make_async_copy(src, dst, sem1.at[i]).wait()
        s_ref[:, :hdim] = jnp.dot(x_ref[...], w1_v[...],
                                  preferred_element_type=jnp.float32)
        s_ref[:, hdim:] = jnp.dot(x_ref[...], w11_v[...],
                                  preferred_element_type=jnp.float32)

    @pl.when(t < nj)
    def _():
        y = jnp.dot(adj_ref[...], s_ref[...],
                    preferred_element_type=jnp.float32)
        y1 = jnp.maximum(y[:, :hdim] + b1_v[...], 0.0)
        y2 = jnp.maximum(y[:, hdim:] + b11_v[...], 0.0)
        h_ref[pl.ds(t * bm, bm), :] = y1 * y2

    @pl.when(t == nj - 1)
    def _():
        for src, dst, i in layer2:
            pltpu.make_async_copy(src, dst, sem2.at[i]).wait()
        t_ref[:, :cdim] = jnp.dot(h_ref[...], w2_v[...],
                                  preferred_element_type=jnp.float32)
        t_ref[:, cdim:] = jnp.dot(h_ref[...], w21_v[...],
                                  preferred_element_type=jnp.float32)

    @pl.when(t >= nj - 1)
    def _():
        y = jnp.dot(adj_ref[...], t_ref[...],
                    preferred_element_type=jnp.float32)
        v = (y[:, :cdim] + b2_v[...]) * (y[:, cdim:] + b21_v[...])
        m = jnp.max(v, axis=1, keepdims=True)
        e = jnp.exp(v - m)
        o_ref[...] = (v - m) - jnp.log(jnp.sum(e, axis=1, keepdims=True))


def kernel(x, adj, W1, b1, W1_1, b1_1, W2, b2, W2_1, b2_1):
    n, nfeat = x.shape
    nhid = W1.shape[1]
    nclass = W2.shape[1]
    bm = _pick_block(n)
    nj = n // bm

    def full(shape):
        return pl.BlockSpec(shape, lambda t: (0,) * len(shape))

    anyspace = pl.BlockSpec(memory_space=pl.ANY)

    out = pl.pallas_call(
        functools.partial(_body, bm=bm, nj=nj, hdim=nhid, cdim=nclass),
        grid=(2 * nj - 1,),
        in_specs=[
            full((n, nfeat)),
            pl.BlockSpec((bm, n), lambda t: (t % nj, 0)),
            anyspace, anyspace, anyspace, anyspace,
            anyspace, anyspace, anyspace, anyspace,
        ],
        out_specs=pl.BlockSpec((bm, nclass), lambda t: (t % nj, 0)),
        out_shape=jax.ShapeDtypeStruct((n, nclass), jnp.float32),
        scratch_shapes=[
            pltpu.VMEM((n, 2 * nhid), jnp.float32),
            pltpu.VMEM((n, 2 * nclass), jnp.float32),
            pltpu.VMEM((n, nhid), jnp.float32),
            pltpu.VMEM((nfeat, nhid), jnp.float32),
            pltpu.VMEM((nhid,), jnp.float32),
            pltpu.VMEM((nfeat, nhid), jnp.float32),
            pltpu.VMEM((nhid,), jnp.float32),
            pltpu.VMEM((nhid, nclass), jnp.float32),
            pltpu.VMEM((nclass,), jnp.float32),
            pltpu.VMEM((nhid, nclass), jnp.float32),
            pltpu.VMEM((nclass,), jnp.float32),
            pltpu.SemaphoreType.DMA((4,)),
            pltpu.SemaphoreType.DMA((4,)),
        ],
        compiler_params=pltpu.CompilerParams(
            dimension_semantics=("arbitrary",),
            vmem_limit_bytes=120 * 1024 * 1024),
    )(x, adj, W1, b1, W1_1, b1_1, W2, b2, W2_1, b2_1)

    return out


# speedup vs baseline: 1.0217x; 1.0217x over previous
import functools

import jax
import jax.numpy as jnp
from jax.experimental import pallas as pl
from jax.experimental.pallas import tpu as pltpu


def _pick_block(n, target=512):
    for bm in (512, 400, 256, 200, 128, 80, 64, 40, 32, 16, 8):
        if bm <= target and n % bm == 0:
            return bm
    return n


def _body(x_ref, adj_ref, w1_ref, b1_ref, w11_ref, b11_ref,
          w2_ref, b2_ref, w21_ref, b21_ref, o_ref,
          s_ref, t_ref, h_ref, *, bm, nj, hdim, cdim):
    t = pl.program_id(0)

    @pl.when(t == 0)
    def _():
        s_ref[:, :hdim] = jnp.dot(x_ref[...], w1_ref[...],
                                  preferred_element_type=jnp.float32)
        s_ref[:, hdim:] = jnp.dot(x_ref[...], w11_ref[...],
                                  preferred_element_type=jnp.float32)

    @pl.when(t < nj)
    def _():
        y = jnp.dot(adj_ref[...], s_ref[...],
                    preferred_element_type=jnp.float32)
        y1 = jnp.maximum(y[:, :hdim] + b1_ref[...], 0.0)
        y2 = jnp.maximum(y[:, hdim:] + b11_ref[...], 0.0)
        h_ref[pl.ds(t * bm, bm), :] = y1 * y2

    @pl.when(t == nj - 1)
    def _():
        t_ref[:, :cdim] = jnp.dot(h_ref[...], w2_ref[...],
                                  preferred_element_type=jnp.float32)
        t_ref[:, cdim:] = jnp.dot(h_ref[...], w21_ref[...],
                                  preferred_element_type=jnp.float32)

    @pl.when(t >= nj - 1)
    def _():
        y = jnp.dot(adj_ref[...], t_ref[...],
                    preferred_element_type=jnp.float32)
        v = (y[:, :cdim] + b2_ref[...]) * (y[:, cdim:] + b21_ref[...])
        m = jnp.max(v, axis=1, keepdims=True)
        e = jnp.exp(v - m)
        ls = (v - m) - jnp.log(jnp.sum(e, axis=1, keepdims=True))
        o_ref[pl.ds((t % nj) * bm, bm), :] = ls


def kernel(x, adj, W1, b1, W1_1, b1_1, W2, b2, W2_1, b2_1):
    n, nfeat = x.shape
    nhid = W1.shape[1]
    nclass = W2.shape[1]
    bm = _pick_block(n)
    nj = n // bm

    hbm = lambda a: pltpu.with_memory_space_constraint(
        a, pltpu.MemorySpace.HBM)
    W1, W1_1, W2, W2_1 = hbm(W1), hbm(W1_1), hbm(W2), hbm(W2_1)
    b1 = hbm(b1[None, :])
    b1_1 = hbm(b1_1[None, :])
    b2 = hbm(b2[None, :])
    b2_1 = hbm(b2_1[None, :])

    def full(shape):
        return pl.BlockSpec(shape, lambda t: (0,) * len(shape))

    out_t = pl.pallas_call(
        functools.partial(_body, bm=bm, nj=nj, hdim=nhid, cdim=nclass),
        grid=(2 * nj - 1,),
        in_specs=[
            full((n, nfeat)),
            pl.BlockSpec((bm, n), lambda t: (t % nj, 0)),
            full((nfeat, nhid)),
            full((1, nhid)),
            full((nfeat, nhid)),
            full((1, nhid)),
            full((nhid, nclass)),
            full((1, nclass)),
            full((nhid, nclass)),
            full((1, nclass)),
        ],
        out_specs=full((n, nclass)),
        out_shape=jax.ShapeDtypeStruct((n, nclass), jnp.float32),
        scratch_shapes=[
            pltpu.VMEM((n, 2 * nhid), jnp.float32),
            pltpu.VMEM((n, 2 * nclass), jnp.float32),
            pltpu.VMEM((n, nhid), jnp.float32),
        ],
        compiler_params=pltpu.CompilerParams(
            dimension_semantics=("arbitrary",),
            vmem_limit_bytes=120 * 1024 * 1024),
    )(x, adj, W1, b1, W1_1, b1_1, W2, b2, W2_1, b2_1)

    return out_t
